# Initial kernel scaffold; baseline (speedup 1.0000x reference)
import jax
import jax.numpy as jnp
from jax.experimental import pallas as pl


def kernel(x, adj, W, b):
    raise NotImplementedError("write your pallas kernel here")


# speedup vs baseline: 60.7882x; 60.7882x over previous
import functools

import jax
import jax.numpy as jnp
from jax.experimental import pallas as pl


def _gcn_kernel(x_ref, adj_ref, w_ref, b_ref, out_ref, *, nk):
    k = pl.program_id(0)
    h = jnp.dot(x_ref[...], w_ref[...], preferred_element_type=jnp.float32)
    contrib = jax.lax.dot_general(
        adj_ref[...], h, (((0,), (0,)), ((), ())),
        preferred_element_type=jnp.float32)

    @pl.when(k == 0)
    def _():
        out_ref[...] = contrib

    @pl.when(k > 0)
    def _():
        out_ref[...] += contrib

    @pl.when(k == nk - 1)
    def _():
        out_ref[...] = jnp.maximum(out_ref[...] + b_ref[...], 0.0)


def kernel(x, adj, W, b):
    n, d_in = x.shape
    d_out = W.shape[1]

    k_tile = 400
    if n % k_tile:
        k_tile = n
    nk = n // k_tile

    b2 = b.reshape(1, d_out).astype(jnp.float32)

    out = pl.pallas_call(
        functools.partial(_gcn_kernel, nk=nk),
        grid=(nk,),
        in_specs=[
            pl.BlockSpec((k_tile, d_in), lambda k: (k, 0)),
            pl.BlockSpec((k_tile, n), lambda k: (k, 0)),
            pl.BlockSpec((d_in, d_out), lambda k: (0, 0)),
            pl.BlockSpec((1, d_out), lambda k: (0, 0)),
        ],
        out_specs=pl.BlockSpec((n, d_out), lambda k: (0, 0)),
        out_shape=jax.ShapeDtypeStruct((n, d_out), jnp.float32),
    )(x, adj, W, b2)
    return (out, adj)


# speedup vs baseline: 62.7455x; 1.0322x over previous
import functools

import jax
import jax.numpy as jnp
from jax.experimental import pallas as pl


def _gcn_kernel(x_ref, adj_ref, w_ref, b_ref, out_ref, *, nk):
    k = pl.program_id(0)
    h = jnp.dot(x_ref[...], w_ref[...], preferred_element_type=jnp.float32)
    contrib = jax.lax.dot_general(
        adj_ref[...].astype(jnp.bfloat16), h.astype(jnp.bfloat16),
        (((0,), (0,)), ((), ())),
        preferred_element_type=jnp.float32)

    @pl.when(k == 0)
    def _():
        out_ref[...] = contrib

    @pl.when(k > 0)
    def _():
        out_ref[...] += contrib

    @pl.when(k == nk - 1)
    def _():
        out_ref[...] = jnp.maximum(out_ref[...] + b_ref[...], 0.0)


def kernel(x, adj, W, b):
    n, d_in = x.shape
    d_out = W.shape[1]

    k_tile = 400
    if n % k_tile:
        k_tile = n
    nk = n // k_tile

    b2 = b.reshape(1, d_out).astype(jnp.float32)

    out = pl.pallas_call(
        functools.partial(_gcn_kernel, nk=nk),
        grid=(nk,),
        in_specs=[
            pl.BlockSpec((k_tile, d_in), lambda k: (k, 0)),
            pl.BlockSpec((k_tile, n), lambda k: (k, 0)),
            pl.BlockSpec((d_in, d_out), lambda k: (0, 0)),
            pl.BlockSpec((1, d_out), lambda k: (0, 0)),
        ],
        out_specs=pl.BlockSpec((n, d_out), lambda k: (0, 0)),
        out_shape=jax.ShapeDtypeStruct((n, d_out), jnp.float32),
    )(x, adj, W, b2)
    return (out, adj)
